# baseline (speedup 1.0000x reference)
import functools

import jax
import jax.numpy as jnp
from jax import lax
from jax.experimental import pallas as pl
from jax.experimental.pallas import tpu as pltpu
from jax.experimental.pallas import tpu_sc as plsc

_D = 1024
_LANES = 16
_NC, _NS = 2, 16
_NW = _NC * _NS
_CHUNK = 32


def _sc_add_pe(x2d, seg1d, pe):
    n = x2d.shape[0]
    per_w = n // _NW
    steps = per_w // _CHUNK
    mesh = plsc.VectorSubcoreMesh(core_axis_name="c", subcore_axis_name="s")

    @functools.partial(
        pl.kernel,
        mesh=mesh,
        out_type=jax.ShapeDtypeStruct((n, _D), jnp.float32),
        scratch_types=[
            pltpu.VMEM((per_w,), jnp.int32),
            pltpu.VMEM((_CHUNK, _D), jnp.float32),
            pltpu.VMEM((_CHUNK, _D), jnp.float32),
            pltpu.SemaphoreType.DMA,
        ],
    )
    def k(x_hbm, seg_hbm, pe_hbm, out_hbm, idx_v, rows_v, x_v, gsem):
        wid = lax.axis_index("s") * _NC + lax.axis_index("c")
        base = wid * per_w
        pltpu.sync_copy(seg_hbm.at[pl.ds(base, per_w)], idx_v)

        @pl.loop(0, steps)
        def _chunk(c):
            row0 = c * _CHUNK
            gcopy = pltpu.async_copy(
                pe_hbm.at[idx_v.at[pl.ds(row0, _CHUNK)]], rows_v, gsem)
            pltpu.sync_copy(x_hbm.at[pl.ds(base + row0, _CHUNK)], x_v)
            gcopy.wait()

            @pl.loop(0, _CHUNK)
            def _row(r):
                for j in range(_D // _LANES):
                    sl = pl.ds(j * _LANES, _LANES)
                    plsc.addupdate(rows_v.at[r, sl], x_v[r, sl])

            pltpu.sync_copy(rows_v, out_hbm.at[pl.ds(base + row0, _CHUNK)])

    return k(x2d, seg1d, pe)


def kernel(x, segment, pe):
    b, s, d = x.shape
    out = _sc_add_pe(x.reshape(b * s, d), segment.reshape(b * s), pe)
    return out.reshape(b, s, d)


# speedup vs baseline: 1.3556x; 1.3556x over previous
import functools

import jax
import jax.numpy as jnp
from jax import lax
from jax.experimental import pallas as pl
from jax.experimental.pallas import tpu as pltpu
from jax.experimental.pallas import tpu_sc as plsc

_D = 1024
_LANES = 16
_NC, _NS = 2, 16
_NW = _NC * _NS
_CHUNK = 16


def _sc_add_pe(x2d, seg1d, pe):
    n = x2d.shape[0]
    per_w = n // _NW
    steps = per_w // _CHUNK
    mesh = plsc.VectorSubcoreMesh(core_axis_name="c", subcore_axis_name="s")

    @functools.partial(
        pl.kernel,
        mesh=mesh,
        out_type=jax.ShapeDtypeStruct((n, _D), jnp.float32),
        scratch_types=[
            pltpu.VMEM((per_w,), jnp.int32),
            pltpu.VMEM((_CHUNK, _D), jnp.float32),
            pltpu.VMEM((_CHUNK, _D), jnp.float32),
            pltpu.VMEM((_CHUNK, _D), jnp.float32),
            pltpu.VMEM((_CHUNK, _D), jnp.float32),
            pltpu.SemaphoreType.DMA,
            pltpu.SemaphoreType.DMA,
            pltpu.SemaphoreType.DMA,
            pltpu.SemaphoreType.DMA,
            pltpu.SemaphoreType.DMA,
            pltpu.SemaphoreType.DMA,
        ],
    )
    def k(x_hbm, seg_hbm, pe_hbm, out_hbm,
          idx_v, rows0, rows1, xv0, xv1,
          gs0, gs1, xs0, xs1, os0, os1):
        wid = lax.axis_index("s") * _NC + lax.axis_index("c")
        base = wid * per_w
        pltpu.sync_copy(seg_hbm.at[pl.ds(base, per_w)], idx_v)

        rows = (rows0, rows1)
        xvs = (xv0, xv1)
        gss = (gs0, gs1)
        xss = (xs0, xs1)
        oss = (os0, os1)

        def gather_desc(c, b):
            return pltpu.make_async_copy(
                pe_hbm.at[idx_v.at[pl.ds(c * _CHUNK, _CHUNK)]], rows[b], gss[b])

        def xin_desc(c, b):
            return pltpu.make_async_copy(
                x_hbm.at[pl.ds(base + c * _CHUNK, _CHUNK)], xvs[b], xss[b])

        def out_desc(c, b):
            return pltpu.make_async_copy(
                rows[b], out_hbm.at[pl.ds(base + c * _CHUNK, _CHUNK)], oss[b])

        def issue_in(c, b):
            gather_desc(c, b).start()
            xin_desc(c, b).start()

        issue_in(0, 0)

        @pl.loop(0, steps, step=2)
        def _pair(c):
            for b in range(2):
                cc = c + b
                gather_desc(cc, b).wait()
                xin_desc(cc, b).wait()

                @pl.when(cc >= 1)
                def _():
                    out_desc(cc - 1, 1 - b).wait()

                @pl.when(cc + 1 < steps)
                def _():
                    issue_in(cc + 1, 1 - b)

                @pl.loop(0, _CHUNK)
                def _row(r):
                    for j in range(_D // _LANES):
                        sl = pl.ds(j * _LANES, _LANES)
                        plsc.addupdate(rows[b].at[r, sl], xvs[b][r, sl])

                out_desc(cc, b).start()

        out_desc(steps - 1, (steps - 1) % 2).wait()

    return k(x2d, seg1d, pe)


def kernel(x, segment, pe):
    b, s, d = x.shape
    out = _sc_add_pe(x.reshape(b * s, d), segment.reshape(b * s), pe)
    return out.reshape(b, s, d)


# speedup vs baseline: 2.3972x; 1.7684x over previous
import functools

import jax
import jax.numpy as jnp
from jax import lax
from jax.experimental import pallas as pl
from jax.experimental.pallas import tpu as pltpu
from jax.experimental.pallas import tpu_sc as plsc

_D = 1024
_LANES = 16
_NC, _NS = 2, 16
_NW = _NC * _NS
_CHUNK = 16


def _sc_add_pe(x2d, seg1d, pe):
    n = x2d.shape[0]
    per_w = n // _NW
    steps = per_w // _CHUNK
    mesh = plsc.VectorSubcoreMesh(core_axis_name="c", subcore_axis_name="s")

    @functools.partial(
        pl.kernel,
        mesh=mesh,
        out_type=jax.ShapeDtypeStruct((n, _D), jnp.float32),
        scratch_types=[
            pltpu.VMEM((per_w,), jnp.int32),
            pltpu.VMEM((_CHUNK, _D), jnp.float32),
            pltpu.VMEM((_CHUNK, _D), jnp.float32),
            pltpu.VMEM((_CHUNK, _D), jnp.float32),
            pltpu.VMEM((_CHUNK, _D), jnp.float32),
            pltpu.SemaphoreType.DMA,
            pltpu.SemaphoreType.DMA,
            pltpu.SemaphoreType.DMA,
            pltpu.SemaphoreType.DMA,
            pltpu.SemaphoreType.DMA,
            pltpu.SemaphoreType.DMA,
        ],
    )
    def k(x_hbm, seg_hbm, pe_hbm, out_hbm,
          idx_v, rows0, rows1, xv0, xv1,
          gs0, gs1, xs0, xs1, os0, os1):
        wid = lax.axis_index("s") * _NC + lax.axis_index("c")
        base = wid * per_w
        pltpu.sync_copy(seg_hbm.at[pl.ds(base, per_w)], idx_v)

        rows = (rows0, rows1)
        xvs = (xv0, xv1)
        gss = (gs0, gs1)
        xss = (xs0, xs1)
        oss = (os0, os1)

        def gather_desc(c, b):
            return pltpu.make_async_copy(
                pe_hbm.at[idx_v.at[pl.ds(c * _CHUNK, _CHUNK)]], rows[b], gss[b])

        def xin_desc(c, b):
            return pltpu.make_async_copy(
                x_hbm.at[pl.ds(base + c * _CHUNK, _CHUNK)], xvs[b], xss[b])

        def out_desc(c, b):
            return pltpu.make_async_copy(
                rows[b], out_hbm.at[pl.ds(base + c * _CHUNK, _CHUNK)], oss[b])

        def issue_in(c, b):
            gather_desc(c, b).start()
            xin_desc(c, b).start()

        issue_in(0, 0)

        @pl.loop(0, steps, step=2)
        def _pair(c):
            for b in range(2):
                cc = c + b
                gather_desc(cc, b).wait()
                xin_desc(cc, b).wait()

                @pl.when(cc >= 1)
                def _():
                    out_desc(cc - 1, 1 - b).wait()

                @pl.when(cc + 1 < steps)
                def _():
                    issue_in(cc + 1, 1 - b)

                @plsc.parallel_loop(0, _CHUNK * (_D // _LANES), unroll=8)
                def _pair_add(t):
                    r = lax.shift_right_logical(t, 6)
                    col = pl.multiple_of(
                        lax.shift_left(lax.bitwise_and(t, _D // _LANES - 1), 4),
                        _LANES)
                    sl = pl.ds(col, _LANES)
                    plsc.addupdate(rows[b].at[r, sl], xvs[b][r, sl])

                out_desc(cc, b).start()

        out_desc(steps - 1, (steps - 1) % 2).wait()

    return k(x2d, seg1d, pe)


def kernel(x, segment, pe):
    b, s, d = x.shape
    out = _sc_add_pe(x.reshape(b * s, d), segment.reshape(b * s), pe)
    return out.reshape(b, s, d)


# speedup vs baseline: 2.5197x; 1.0511x over previous
import functools

import jax
import jax.numpy as jnp
from jax import lax
from jax.experimental import pallas as pl
from jax.experimental.pallas import tpu as pltpu
from jax.experimental.pallas import tpu_sc as plsc

_D = 1024
_LANES = 16
_NC, _NS = 2, 16
_NW = _NC * _NS
_CHUNK = 8
_RING = 4


def _sc_add_pe(x2d, seg1d, pe):
    n = x2d.shape[0]
    per_w = n // _NW
    steps = per_w // _CHUNK
    mesh = plsc.VectorSubcoreMesh(core_axis_name="c", subcore_axis_name="s")

    @functools.partial(
        pl.kernel,
        mesh=mesh,
        out_type=jax.ShapeDtypeStruct((n, _D), jnp.float32),
        scratch_types=(
            [pltpu.VMEM((per_w,), jnp.int32)]
            + [pltpu.VMEM((_CHUNK, _D), jnp.float32) for _ in range(2 * _RING)]
            + [pltpu.SemaphoreType.DMA for _ in range(3 * _RING)]
        ),
    )
    def k(x_hbm, seg_hbm, pe_hbm, out_hbm, idx_v, *bufs):
        rows = bufs[:_RING]
        xvs = bufs[_RING:2 * _RING]
        gss = bufs[2 * _RING:3 * _RING]
        xss = bufs[3 * _RING:4 * _RING]
        oss = bufs[4 * _RING:5 * _RING]

        wid = lax.axis_index("s") * _NC + lax.axis_index("c")
        base = wid * per_w
        pltpu.sync_copy(seg_hbm.at[pl.ds(base, per_w)], idx_v)

        def gather_desc(c, b):
            return pltpu.make_async_copy(
                pe_hbm.at[idx_v.at[pl.ds(c * _CHUNK, _CHUNK)]], rows[b], gss[b])

        def xin_desc(c, b):
            return pltpu.make_async_copy(
                x_hbm.at[pl.ds(base + c * _CHUNK, _CHUNK)], xvs[b], xss[b])

        def out_desc(c, b):
            return pltpu.make_async_copy(
                rows[b], out_hbm.at[pl.ds(base + c * _CHUNK, _CHUNK)], oss[b])

        def issue_in(c, b):
            gather_desc(c, b).start()
            xin_desc(c, b).start()

        for c0 in range(_RING - 1):
            issue_in(c0, c0)

        @pl.loop(0, steps, step=_RING)
        def _group(c):
            for b in range(_RING):
                cc = c + b
                bprev = (b - 1) % _RING
                gather_desc(cc, b).wait()
                xin_desc(cc, b).wait()

                @pl.when(cc >= 1)
                def _():
                    out_desc(cc - 1, bprev).wait()

                @pl.when(cc + _RING - 1 < steps)
                def _():
                    issue_in(cc + _RING - 1, bprev)

                @plsc.parallel_loop(0, _CHUNK * (_D // _LANES), unroll=16)
                def _pair_add(t):
                    r = lax.shift_right_logical(t, 6)
                    col = pl.multiple_of(
                        lax.shift_left(lax.bitwise_and(t, _D // _LANES - 1), 4),
                        _LANES)
                    sl = pl.ds(col, _LANES)
                    plsc.addupdate(rows[b].at[r, sl], xvs[b][r, sl])

                out_desc(cc, b).start()

        out_desc(steps - 1, (steps - 1) % _RING).wait()

    return k(x2d, seg1d, pe)


def kernel(x, segment, pe):
    b, s, d = x.shape
    out = _sc_add_pe(x.reshape(b * s, d), segment.reshape(b * s), pe)
    return out.reshape(b, s, d)
